# Initial kernel scaffold; baseline (speedup 1.0000x reference)
import jax
import jax.numpy as jnp
from jax.experimental import pallas as pl


def kernel(inp, table):
    raise NotImplementedError("write your pallas kernel here")


# speedup vs baseline: 1.3079x; 1.3079x over previous
import functools

import jax
import jax.numpy as jnp
from jax import lax
from jax.experimental import pallas as pl
from jax.experimental.pallas import tpu as pltpu
from jax.experimental.pallas import tpu_sc as plsc

VOCAB = 1000000
DIM = 32
B = 4096
L = 200

NUM_WORKERS = 32
CHUNK = 128
N_FLAT = B * L
PER_WORKER = N_FLAT // NUM_WORKERS
CHUNKS_PER_WORKER = PER_WORKER // CHUNK


def _make_sc_gather():
  mesh = plsc.VectorSubcoreMesh(core_axis_name="c", subcore_axis_name="s")

  @functools.partial(
      pl.kernel,
      mesh=mesh,
      out_type=jax.ShapeDtypeStruct((N_FLAT, DIM), jnp.float32),
      compiler_params=pltpu.CompilerParams(use_tc_tiling_on_sc=False),
      scratch_types=[
          pltpu.VMEM((CHUNKS_PER_WORKER, CHUNK), jnp.int32),
          pltpu.VMEM((CHUNK, DIM), jnp.float32),
          pltpu.SemaphoreType.DMA,
      ],
  )
  def gather_kernel(table_hbm, idx_hbm, out_hbm, idx_v, rows_v, sem):
    wid = lax.axis_index("s") * 2 + lax.axis_index("c")
    chunk_base = wid * CHUNKS_PER_WORKER
    pltpu.sync_copy(idx_hbm.at[pl.ds(chunk_base, CHUNKS_PER_WORKER)], idx_v)

    def body(j, carry):
      pltpu.async_copy(table_hbm.at[idx_v.at[j]], rows_v, sem).wait()
      row0 = (chunk_base + j) * CHUNK
      pltpu.sync_copy(rows_v, out_hbm.at[pl.ds(row0, CHUNK)])
      return carry

    lax.fori_loop(0, CHUNKS_PER_WORKER, body, 0)

  return gather_kernel


_sc_gather = _make_sc_gather()


def kernel(inp, table):
  idx = inp.astype(jnp.int32).reshape(N_FLAT // CHUNK, CHUNK)
  out = _sc_gather(table, idx)
  return out.reshape(B, L, DIM)


# speedup vs baseline: 1.4948x; 1.1429x over previous
import functools

import jax
import jax.numpy as jnp
from jax import lax
from jax.experimental import pallas as pl
from jax.experimental.pallas import tpu as pltpu
from jax.experimental.pallas import tpu_sc as plsc

VOCAB = 1000000
DIM = 32
B = 4096
L = 200

NUM_WORKERS = 32
CHUNK = 128
N_FLAT = B * L
PER_WORKER = N_FLAT // NUM_WORKERS
CHUNKS_PER_WORKER = PER_WORKER // CHUNK


K = 10
NT = CHUNKS_PER_WORKER // (2 * K)


def _make_sc_gather():
  mesh = plsc.VectorSubcoreMesh(core_axis_name="c", subcore_axis_name="s")

  @functools.partial(
      pl.kernel,
      mesh=mesh,
      out_type=jax.ShapeDtypeStruct((N_FLAT, DIM), jnp.float32),
      compiler_params=pltpu.CompilerParams(use_tc_tiling_on_sc=False),
      scratch_types=[
          pltpu.VMEM((CHUNKS_PER_WORKER, CHUNK), jnp.int32),
          pltpu.VMEM((2, K, CHUNK, DIM), jnp.float32),
          pltpu.SemaphoreType.DMA,
          pltpu.SemaphoreType.DMA,
          pltpu.SemaphoreType.DMA,
          pltpu.SemaphoreType.DMA,
      ],
  )
  def gather_kernel(table_hbm, idx_hbm, out_hbm, idx_v, rows_v, ga, gb, sa, sb):
    wid = lax.axis_index("s") * 2 + lax.axis_index("c")
    chunk_base = wid * CHUNKS_PER_WORKER
    pltpu.sync_copy(idx_hbm.at[pl.ds(chunk_base, CHUNKS_PER_WORKER)], idx_v)

    def gath(j, buf_half, b, sem):
      return pltpu.make_async_copy(
          table_hbm.at[idx_v.at[j]], rows_v.at[buf_half, b], sem)

    def store(j, buf_half, b, sem):
      row0 = (chunk_base + j) * CHUNK
      return pltpu.make_async_copy(
          rows_v.at[buf_half, b], out_hbm.at[pl.ds(row0, CHUNK)], sem)

    for b in range(K):
      gath(b, 0, b, ga).start()

    def body(t, carry):
      base = t * 2 * K
      for b in range(K):
        gath(base + b, 0, b, ga).wait()
      @pl.when(t > 0)
      def _():
        for b in range(K):
          store(base - K + b, 1, b, sb).wait()
      for b in range(K):
        gath(base + K + b, 1, b, gb).start()
      for b in range(K):
        store(base + b, 0, b, sa).start()
      for b in range(K):
        gath(base + K + b, 1, b, gb).wait()
      for b in range(K):
        store(base + b, 0, b, sa).wait()
      @pl.when(t < NT - 1)
      def _():
        for b in range(K):
          gath(base + 2 * K + b, 0, b, ga).start()
      for b in range(K):
        store(base + K + b, 1, b, sb).start()
      return carry

    lax.fori_loop(0, NT, body, 0)
    last = (NT - 1) * 2 * K + K
    for b in range(K):
      store(last + b, 1, b, sb).wait()

  return gather_kernel


_sc_gather = _make_sc_gather()


def kernel(inp, table):
  idx = inp.astype(jnp.int32).reshape(N_FLAT // CHUNK, CHUNK)
  out = _sc_gather(table, idx)
  return out.reshape(B, L, DIM)
